# Initial kernel scaffold; baseline (speedup 1.0000x reference)
import jax
import jax.numpy as jnp
from jax.experimental import pallas as pl


def kernel(x, cluster, Wq, bq, Wk, bk, Wv, bv, Wg, bg, Wo, bo):
    raise NotImplementedError("write your pallas kernel here")


# speedup vs baseline: 4.5398x; 4.5398x over previous
import functools
import numpy as np
import jax
import jax.numpy as jnp
from jax.experimental import pallas as pl
from jax.experimental.pallas import tpu as pltpu

SEG = 8
NVAR = 100
H = 16
D = 1024
G = 10
K = 10
DH = D // H
R = H * G
SCALE = float((D / H) ** -0.5)


def _q_kernel(cl_ref, wq_ref, bq_ref, qblk_ref):
    qq = jnp.dot(cl_ref[0], wq_ref[...], preferred_element_type=jnp.float32)
    qq = qq + bq_ref[...]
    q3 = jnp.broadcast_to(qq[None, :, :], (H, G, D))
    h_iota = jax.lax.broadcasted_iota(jnp.int32, (H, G, D), 0)
    d_iota = jax.lax.broadcasted_iota(jnp.int32, (H, G, D), 2)
    qblk_ref[0] = jnp.where(d_iota // DH == h_iota, q3, 0.0).reshape(R, D)


def _main_kernel(x_ref, qblk_ref, wk_ref, bk_ref, wv_ref, bv_ref,
                 wgr_ref, bgr_ref, sel_ref, wo_ref, bo_ref,
                 out_ref, attn_ref, idx_ref):
    a = x_ref[0]
    a = jnp.log(jnp.maximum(a, 0.0) + 1.0)
    kb = jnp.dot(a, wk_ref[...], preferred_element_type=jnp.float32) + bk_ref[...]
    vb = jnp.dot(a, wv_ref[...], preferred_element_type=jnp.float32) + bv_ref[...]
    qblk = qblk_ref[0]
    scores = jax.lax.dot_general(
        qblk, kb, (((1,), (1,)), ((), ())),
        preferred_element_type=jnp.float32) * SCALE

    lane = jax.lax.broadcasted_iota(jnp.int32, (R, NVAR), 1)
    s = scores
    vals, idxs = [], []
    for _ in range(K):
        m = jnp.max(s, axis=1, keepdims=True)
        cand = jnp.where(s == m, lane, jnp.int32(NVAR))
        j = jnp.min(cand, axis=1, keepdims=True)
        vals.append(m)
        idxs.append(j)
        s = jnp.where(lane == j, -jnp.inf, s)
    topv = jnp.concatenate(vals, axis=1)
    topi = jnp.concatenate(idxs, axis=1)

    e = jnp.exp(topv - topv[:, 0:1])
    attn_ref[0] = e / jnp.sum(e, axis=1, keepdims=True)
    idx_ref[0] = topi

    wsel = jnp.zeros((R, NVAR), jnp.float32)
    for r in range(K):
        wsel = wsel + jnp.where(lane == topi[:, r:r + 1],
                                wgr_ref[:, r:r + 1], 0.0)
    xg_full = jnp.dot(wsel, vb, preferred_element_type=jnp.float32)

    rowD = jax.lax.broadcasted_iota(jnp.int32, (R, D), 0)
    colD = jax.lax.broadcasted_iota(jnp.int32, (R, D), 1)
    keep = (colD // DH) == (rowD // G)
    xgb = xg_full + bgr_ref[...]
    x_max = jnp.max(jnp.where(keep, xgb, -jnp.inf), axis=1, keepdims=True)
    x_min = jnp.min(jnp.where(keep, xgb, jnp.inf), axis=1, keepdims=True)
    denom = jnp.maximum(x_max - x_min, 1e-6)
    z = jnp.where(keep, jnp.exp((xgb - x_min) / denom), 0.0)
    xo = jnp.dot(sel_ref[...], z, preferred_element_type=jnp.float32)
    out_ref[0] = jnp.dot(xo, wo_ref[...],
                         preferred_element_type=jnp.float32) + bo_ref[...]


@jax.jit
def _run(x, cluster, Wq, bq, Wk, bk, Wv, bv, Wg, bg, Wo, bo):
    bs = x.shape[0]
    nb = bs // SEG

    qblk = pl.pallas_call(
        _q_kernel,
        grid=(SEG,),
        in_specs=[
            pl.BlockSpec((1, G, D), lambda s: (s, 0, 0)),
            pl.BlockSpec((D, D), lambda s: (0, 0)),
            pl.BlockSpec((1, D), lambda s: (0, 0)),
        ],
        out_specs=pl.BlockSpec((1, R, D), lambda s: (s, 0, 0)),
        out_shape=jax.ShapeDtypeStruct((SEG, R, D), jnp.float32),
    )(cluster, Wq, bq.reshape(1, D))

    wg_rows = jnp.tile(Wg, (H, 1))
    bg_rows = jnp.tile(bg.reshape(G, 1), (H, 1))
    hg = np.arange(R)
    sel = jnp.asarray((hg[None, :] % G == np.arange(G)[:, None])
                      .astype(np.float32))

    out, attn_t, idx_t = pl.pallas_call(
        _main_kernel,
        grid=(SEG, nb),
        in_specs=[
            pl.BlockSpec((1, NVAR, D), lambda s, b: (b * SEG + s, 0, 0)),
            pl.BlockSpec((1, R, D), lambda s, b: (s, 0, 0)),
            pl.BlockSpec((D, D), lambda s, b: (0, 0)),
            pl.BlockSpec((1, D), lambda s, b: (0, 0)),
            pl.BlockSpec((D, D), lambda s, b: (0, 0)),
            pl.BlockSpec((1, D), lambda s, b: (0, 0)),
            pl.BlockSpec((R, K), lambda s, b: (0, 0)),
            pl.BlockSpec((R, 1), lambda s, b: (0, 0)),
            pl.BlockSpec((G, R), lambda s, b: (0, 0)),
            pl.BlockSpec((D, D), lambda s, b: (0, 0)),
            pl.BlockSpec((1, D), lambda s, b: (0, 0)),
        ],
        out_specs=[
            pl.BlockSpec((1, G, D), lambda s, b: (b * SEG + s, 0, 0)),
            pl.BlockSpec((1, R, K), lambda s, b: (b * SEG + s, 0, 0)),
            pl.BlockSpec((1, R, K), lambda s, b: (b * SEG + s, 0, 0)),
        ],
        out_shape=[
            jax.ShapeDtypeStruct((bs, G, D), jnp.float32),
            jax.ShapeDtypeStruct((bs, R, K), jnp.float32),
            jax.ShapeDtypeStruct((bs, R, K), jnp.int32),
        ],
    )(x, qblk, Wk, bk.reshape(1, D), Wv, bv.reshape(1, D),
      wg_rows, bg_rows, sel, Wo, bo.reshape(1, D))

    attn_k = attn_t.reshape(bs, H, G, K)
    idx = idx_t.reshape(bs, H, G, K)
    return out, attn_k, idx


def kernel(x, cluster, Wq, bq, Wk, bk, Wv, bv, Wg, bg, Wo, bo):
    return _run(x, cluster, Wq, bq, Wk, bk, Wv, bv, Wg, bg, Wo, bo)
